# Initial kernel scaffold; baseline (speedup 1.0000x reference)
import jax
import jax.numpy as jnp
from jax.experimental import pallas as pl


def kernel(x, mu_pop, L_pop, mu_subj, L_subj, gamma, subject_ids):
    raise NotImplementedError("write your pallas kernel here")


# speedup vs baseline: 14.7119x; 14.7119x over previous
import jax
import jax.numpy as jnp
import numpy as np
from jax.experimental import pallas as pl
from jax.experimental.pallas import tpu as pltpu

_LAMBDA_MU = 0.1
_LAMBDA_L = 0.1
_N_SUBJECTS = 16
_LOG2PI = float(np.log(2.0 * np.pi))


def _body(sid_ref, ids_ref, x_ref, gT_ref, mu_subj_ref, mu_pop_ref,
          UsT_ref, UpT_ref, out_ref, invT_scr, mubar_scr, lds_scr):
    B, T, D, K, P = 16, 512, 32, 8, 16
    f32 = jnp.float32

    U = UsT_ref[...]
    ii = jax.lax.broadcasted_iota(jnp.int32, (D, D), 0)
    jj = jax.lax.broadcasted_iota(jnp.int32, (D, D), 1)
    eye = (ii == jj).astype(f32)
    strict_up = (jj > ii).astype(f32)

    d = jnp.sum(U * eye, axis=-1)
    N = -(U * strict_up) / d[:, :, None]
    bmm = lambda a, b: jax.lax.dot_general(
        a, b, (((2,), (1,)), ((0,), (0,))), preferred_element_type=f32)
    X = eye[None] + N
    Npow = N
    for _ in range(4):
        Npow = bmm(Npow, Npow)
        X = X + bmm(X, Npow)
    invT = X / d[:, None, :]
    invT_scr[...] = invT

    mu = mu_subj_ref[...]
    mubar_scr[...] = jnp.sum(mu[:, :, None] * invT, axis=1).reshape(P, K, D)
    lds_scr[...] = jnp.sum(jnp.log(d), axis=-1).reshape(P, K)

    acc = jnp.zeros((), dtype=f32)
    for b in range(B):
        s = sid_ref[b]
        Xc = x_ref[b]
        W = invT_scr[pl.ds(s * K, K)]
        mb = mubar_scr[s]
        ld = lds_scr[pl.ds(s, 1)]
        Yb = bmm(jnp.broadcast_to(Xc[None], (K, T, D)), W)
        Ymm = Yb - mb[:, None, :]
        q = jnp.sum(Ymm * Ymm, axis=2)
        logp = (-0.5 * D * _LOG2PI) - ld.reshape(K, 1) - 0.5 * q
        acc = acc + jnp.sum(gT_ref[b] * logp)
    nll = -acc / float(B * T)

    ids_v = ids_ref[...]
    pio = jax.lax.broadcasted_iota(jnp.int32, (P, B), 0)
    pres = jnp.max((pio == ids_v).astype(f32), axis=1, keepdims=True)
    S = jnp.sum(pres)

    md = mu - mu_pop_ref[...]
    msq = jnp.sum(md * md, axis=1, keepdims=True)
    mu_per = jnp.sum(msq.reshape(P, K), axis=1, keepdims=True)
    mu_reg = jnp.sum(pres * mu_per)

    Ld = U - UpT_ref[...]
    lsq = jnp.sum(jnp.sum(Ld * Ld, axis=2), axis=1)
    L_per = jnp.sum(lsq.reshape(P, K), axis=1, keepdims=True)
    L_reg = jnp.sum(pres * L_per)

    reg = (S / _N_SUBJECTS) * (
        _LAMBDA_MU / 2.0 * mu_reg + _LAMBDA_L / 2.0 * L_reg)
    out_ref[0, 0] = nll + reg


def kernel(x, mu_pop, L_pop, mu_subj, L_subj, gamma, subject_ids):
    B, T, D = x.shape
    K = mu_pop.shape[0]
    P = mu_subj.shape[0]
    f32 = jnp.float32

    sid = subject_ids.astype(jnp.int32)
    gT = jnp.swapaxes(gamma, 1, 2)
    UsT = jnp.swapaxes(L_subj.reshape(P * K, D, D), 1, 2)
    UpT = jnp.tile(jnp.swapaxes(L_pop, 1, 2), (P, 1, 1))
    mu_s = mu_subj.reshape(P * K, D)
    mu_p = jnp.tile(mu_pop, (P, 1))
    ids2 = sid.reshape(1, B)

    out = pl.pallas_call(
        _body,
        out_shape=jax.ShapeDtypeStruct((1, 1), f32),
        in_specs=[
            pl.BlockSpec(memory_space=pltpu.SMEM),
            pl.BlockSpec(memory_space=pltpu.VMEM),
            pl.BlockSpec(memory_space=pltpu.VMEM),
            pl.BlockSpec(memory_space=pltpu.VMEM),
            pl.BlockSpec(memory_space=pltpu.VMEM),
            pl.BlockSpec(memory_space=pltpu.VMEM),
            pl.BlockSpec(memory_space=pltpu.VMEM),
            pl.BlockSpec(memory_space=pltpu.VMEM),
        ],
        out_specs=pl.BlockSpec(memory_space=pltpu.SMEM),
        scratch_shapes=[
            pltpu.VMEM((P * K, D, D), f32),
            pltpu.VMEM((P, K, D), f32),
            pltpu.VMEM((P, K), f32),
        ],
    )(sid, ids2, x, gT, mu_s, mu_p, UsT, UpT)
    return out[0, 0]


# speedup vs baseline: 18.2023x; 1.2372x over previous
import jax
import jax.numpy as jnp
import numpy as np
from jax.experimental import pallas as pl
from jax.experimental.pallas import tpu as pltpu

_LAMBDA_MU = 0.1
_LAMBDA_L = 0.1
_N_SUBJECTS = 16
_LOG2PI = float(np.log(2.0 * np.pi))


def _body(sid_ref, ids_ref, xa_ref, xaT_ref, gT_ref, mu_subj_ref,
          mu_pop_ref, Ls_ref, Lp_ref, diag_ref, out_ref, ahat_scr):
    B, T, D, K, P = 16, 512, 32, 8, 16
    DA = D + 1
    f32 = jnp.float32

    L = Ls_ref[...]
    ii = jax.lax.broadcasted_iota(jnp.int32, (D, D), 0)
    jj = jax.lax.broadcasted_iota(jnp.int32, (D, D), 1)
    eye = (ii == jj).astype(f32)
    strict_lo = (ii > jj).astype(f32)

    d = diag_ref[...]
    rinv = 1.0 / d
    N = -(L * strict_lo) * rinv[:, :, None]
    bmm = lambda a, b, dn: jax.lax.dot_general(
        a, b, (dn, ((0,), (0,))), preferred_element_type=f32)
    X = eye[None] + N
    Npow = N
    for _ in range(4):
        Npow = bmm(Npow, Npow, ((2,), (1,)))
        X = X + bmm(X, Npow, ((2,), (1,)))
    Linv = X * rinv[:, None, :]

    A = bmm(Linv, Linv, ((1,), (1,)))
    mu = mu_subj_ref[...]
    bvec = jnp.sum(A * mu[:, None, :], axis=2)
    cval = jnp.sum(mu * bvec, axis=1, keepdims=True)
    logdet = jnp.sum(jnp.log(d), axis=1, keepdims=True)
    Cval = (-0.5 * D * _LOG2PI) - logdet

    ahat_scr[:, 0:D, 0:D] = A
    ahat_scr[:, D:DA, 0:D] = -bvec[:, None, :]
    ahat_scr[:, 0:D, D:DA] = -bvec[:, :, None]
    ahat_scr[:, D:DA, D:DA] = (cval - 2.0 * Cval)[:, :, None]

    acc = jnp.zeros((), dtype=f32)
    for b in range(B):
        s = sid_ref[b]
        gT = gT_ref[b]
        xT = xaT_ref[b]
        xa = xa_ref[b]
        Wg = gT[:, None, :] * xT[None]
        S = bmm(Wg, jnp.broadcast_to(xa[None], (K, T, DA)), ((2,), (1,)))
        Ah = ahat_scr[pl.ds(s * K, K)]
        acc = acc + jnp.sum(Ah * S)
    nll = 0.5 * acc / float(B * T)

    ids_v = ids_ref[...]
    pio = jax.lax.broadcasted_iota(jnp.int32, (P, B), 0)
    pres = jnp.max((pio == ids_v).astype(f32), axis=1, keepdims=True)
    S_cnt = jnp.sum(pres)

    md = mu - mu_pop_ref[...]
    msq = jnp.sum(md * md, axis=1, keepdims=True)
    mu_per = jnp.sum(msq.reshape(P, K), axis=1, keepdims=True)
    mu_reg = jnp.sum(pres * mu_per)

    Ld = L - Lp_ref[...]
    lsq = jnp.sum(jnp.sum(Ld * Ld, axis=2), axis=1)
    L_per = jnp.sum(lsq.reshape(P, K), axis=1, keepdims=True)
    L_reg = jnp.sum(pres * L_per)

    reg = (S_cnt / _N_SUBJECTS) * (
        _LAMBDA_MU / 2.0 * mu_reg + _LAMBDA_L / 2.0 * L_reg)
    out_ref[0, 0] = nll + reg


def kernel(x, mu_pop, L_pop, mu_subj, L_subj, gamma, subject_ids):
    B, T, D = x.shape
    K = mu_pop.shape[0]
    P = mu_subj.shape[0]
    f32 = jnp.float32

    sid = subject_ids.astype(jnp.int32)
    xa = jnp.concatenate([x, jnp.ones((B, T, 1), f32)], axis=2)
    xaT = jnp.swapaxes(xa, 1, 2)
    gT = jnp.swapaxes(gamma, 1, 2)
    Ls = L_subj.reshape(P * K, D, D)
    Lp = jnp.tile(L_pop, (P, 1, 1))
    mu_s = mu_subj.reshape(P * K, D)
    mu_p = jnp.tile(mu_pop, (P, 1))
    diag = jnp.diagonal(Ls, axis1=1, axis2=2)
    ids2 = sid.reshape(1, B)

    out = pl.pallas_call(
        _body,
        out_shape=jax.ShapeDtypeStruct((1, 1), f32),
        in_specs=[
            pl.BlockSpec(memory_space=pltpu.SMEM),
            pl.BlockSpec(memory_space=pltpu.VMEM),
            pl.BlockSpec(memory_space=pltpu.VMEM),
            pl.BlockSpec(memory_space=pltpu.VMEM),
            pl.BlockSpec(memory_space=pltpu.VMEM),
            pl.BlockSpec(memory_space=pltpu.VMEM),
            pl.BlockSpec(memory_space=pltpu.VMEM),
            pl.BlockSpec(memory_space=pltpu.VMEM),
            pl.BlockSpec(memory_space=pltpu.VMEM),
            pl.BlockSpec(memory_space=pltpu.VMEM),
        ],
        out_specs=pl.BlockSpec(memory_space=pltpu.SMEM),
        scratch_shapes=[
            pltpu.VMEM((P * K, D + 1, D + 1), f32),
        ],
    )(sid, ids2, xa, xaT, gT, mu_s, mu_p, Ls, Lp, diag)
    return out[0, 0]


# speedup vs baseline: 20.0591x; 1.1020x over previous
import jax
import jax.numpy as jnp
import numpy as np
from jax.experimental import pallas as pl
from jax.experimental.pallas import tpu as pltpu

_LAMBDA_MU = 0.1
_LAMBDA_L = 0.1
_N_SUBJECTS = 16
_LOG2PI = float(np.log(2.0 * np.pi))


def _body(sid_ref, ids_ref, xa_ref, xaT_ref, gT_ref, mu_subj_ref,
          mu_pop_ref, Ls_ref, Lp_ref, diag_ref, out_ref, ahat_scr):
    B, T, D, K, P = 16, 512, 32, 8, 16
    DA = D + 1
    f32 = jnp.float32

    L = Ls_ref[...]
    ii = jax.lax.broadcasted_iota(jnp.int32, (D, D), 0)
    jj = jax.lax.broadcasted_iota(jnp.int32, (D, D), 1)
    eye = (ii == jj).astype(f32)
    strict_lo = (ii > jj).astype(f32)

    d = diag_ref[...]
    rinv = 1.0 / d
    N = -(L * strict_lo) * rinv[:, :, None]
    bmm = lambda a, b, dn: jax.lax.dot_general(
        a, b, (dn, ((0,), (0,))), preferred_element_type=f32)
    X = eye[None] + N
    Npow = N
    for _ in range(4):
        Npow = bmm(Npow, Npow, ((2,), (1,)))
        X = X + bmm(X, Npow, ((2,), (1,)))
    Linv = X * rinv[:, None, :]

    A = bmm(Linv, Linv, ((1,), (1,)))
    mu = mu_subj_ref[...]
    bvec = jnp.sum(A * mu[:, None, :], axis=2)
    cval = jnp.sum(mu * bvec, axis=1, keepdims=True)
    logdet = jnp.sum(jnp.log(d), axis=1, keepdims=True)
    Cval = (-0.5 * D * _LOG2PI) - logdet

    ahat_scr[:, 0:D, 0:D] = A
    ahat_scr[:, D:DA, 0:D] = -bvec[:, None, :]
    ahat_scr[:, 0:D, D:DA] = -bvec[:, :, None]
    ahat_scr[:, D:DA, D:DA] = (cval - 2.0 * Cval)[:, :, None]

    acc = jnp.zeros((), dtype=f32)
    for b in range(B):
        s = sid_ref[b]
        gT = gT_ref[b]
        xT = xaT_ref[b]
        xa = xa_ref[b]
        Wg = gT[:, None, :] * xT[None]
        S = bmm(Wg, jnp.broadcast_to(xa[None], (K, T, DA)), ((2,), (1,)))
        Ah = ahat_scr[pl.ds(s * K, K)]
        acc = acc + jnp.sum(Ah * S)
    nll = 0.5 * acc / float(B * T)

    ids_v = ids_ref[...]
    pio = jax.lax.broadcasted_iota(jnp.int32, (P, B), 0)
    pres = jnp.max((pio == ids_v).astype(f32), axis=1, keepdims=True)
    S_cnt = jnp.sum(pres)

    md = mu.reshape(P, K, D) - mu_pop_ref[...][None]
    msq = jnp.sum(jnp.sum(md * md, axis=2), axis=1, keepdims=True)
    mu_reg = jnp.sum(pres * msq)

    Ld = L.reshape(P, K, D, D) - Lp_ref[...][None]
    lsq = jnp.sum(jnp.sum(jnp.sum(Ld * Ld, axis=3), axis=2), axis=1,
                  keepdims=True)
    L_reg = jnp.sum(pres * lsq)

    reg = (S_cnt / _N_SUBJECTS) * (
        _LAMBDA_MU / 2.0 * mu_reg + _LAMBDA_L / 2.0 * L_reg)
    out_ref[0, 0] = nll + reg


def kernel(x, mu_pop, L_pop, mu_subj, L_subj, gamma, subject_ids):
    B, T, D = x.shape
    K = mu_pop.shape[0]
    P = mu_subj.shape[0]
    f32 = jnp.float32

    sid = subject_ids.astype(jnp.int32)
    xa = jnp.concatenate([x, jnp.ones((B, T, 1), f32)], axis=2)
    xaT = jnp.swapaxes(xa, 1, 2)
    gT = jnp.swapaxes(gamma, 1, 2)
    Ls = L_subj.reshape(P * K, D, D)
    mu_s = mu_subj.reshape(P * K, D)
    diag = jnp.diagonal(Ls, axis1=1, axis2=2)
    ids2 = sid.reshape(1, B)

    out = pl.pallas_call(
        _body,
        out_shape=jax.ShapeDtypeStruct((1, 1), f32),
        in_specs=[
            pl.BlockSpec(memory_space=pltpu.SMEM),
            pl.BlockSpec(memory_space=pltpu.VMEM),
            pl.BlockSpec(memory_space=pltpu.VMEM),
            pl.BlockSpec(memory_space=pltpu.VMEM),
            pl.BlockSpec(memory_space=pltpu.VMEM),
            pl.BlockSpec(memory_space=pltpu.VMEM),
            pl.BlockSpec(memory_space=pltpu.VMEM),
            pl.BlockSpec(memory_space=pltpu.VMEM),
            pl.BlockSpec(memory_space=pltpu.VMEM),
            pl.BlockSpec(memory_space=pltpu.VMEM),
        ],
        out_specs=pl.BlockSpec(memory_space=pltpu.SMEM),
        scratch_shapes=[
            pltpu.VMEM((P * K, D + 1, D + 1), f32),
        ],
    )(sid, ids2, xa, xaT, gT, mu_s, mu_pop, Ls, L_pop, diag)
    return out[0, 0]


# speedup vs baseline: 28.0508x; 1.3984x over previous
import jax
import jax.numpy as jnp
import numpy as np
from jax.experimental import pallas as pl
from jax.experimental.pallas import tpu as pltpu

_LAMBDA_MU = 0.1
_LAMBDA_L = 0.1
_N_SUBJECTS = 16
_LOG2PI = float(np.log(2.0 * np.pi))


def _body(sid_ref, ids_ref, xgT_ref, mu_subj_ref,
          mu_pop_ref, Ls_ref, Lp_ref, diag_ref, out_ref, ahat_scr):
    B, T, D, K, P = 16, 512, 32, 8, 16
    DA = D + 1
    f32 = jnp.float32

    L = Ls_ref[...]
    ii = jax.lax.broadcasted_iota(jnp.int32, (D, D), 0)
    jj = jax.lax.broadcasted_iota(jnp.int32, (D, D), 1)
    eye = (ii == jj).astype(f32)
    strict_lo = (ii > jj).astype(f32)

    d = diag_ref[...]
    rinv = 1.0 / d
    N = -(L * strict_lo) * rinv[:, :, None]
    bmm = lambda a, b, dn: jax.lax.dot_general(
        a, b, (dn, ((0,), (0,))), preferred_element_type=f32)
    X = eye[None] + N
    Npow = N
    for _ in range(4):
        Npow = bmm(Npow, Npow, ((2,), (1,)))
        X = X + bmm(X, Npow, ((2,), (1,)))
    Linv = X * rinv[:, None, :]

    A = bmm(Linv, Linv, ((1,), (1,)))
    mu = mu_subj_ref[...]
    bvec = jnp.sum(A * mu[:, None, :], axis=2)
    cval = jnp.sum(mu * bvec, axis=1, keepdims=True)
    logdet = jnp.sum(jnp.log(d), axis=1, keepdims=True)
    Cval = (-0.5 * D * _LOG2PI) - logdet

    ahat_scr[:, 0:D, 0:D] = A
    ahat_scr[:, D:DA, 0:D] = -bvec[:, None, :]
    ahat_scr[:, 0:D, D:DA] = -bvec[:, :, None]
    ahat_scr[:, D:DA, D:DA] = (cval - 2.0 * Cval)[:, :, None]

    acc = jnp.zeros((), dtype=f32)
    for b in range(B):
        s = sid_ref[b]
        XG = xgT_ref[b]
        xT = XG[0:DA]
        gT = XG[DA:DA + K]
        Wg = gT[:, None, :] * xT[None]
        S = bmm(Wg, jnp.broadcast_to(xT[None], (K, DA, T)), ((2,), (2,)))
        Ah = ahat_scr[pl.ds(s * K, K)]
        acc = acc + jnp.sum(Ah * S)
    nll = 0.5 * acc / float(B * T)

    ids_v = ids_ref[...]
    pio = jax.lax.broadcasted_iota(jnp.int32, (P, B), 0)
    pres = jnp.max((pio == ids_v).astype(f32), axis=1, keepdims=True)
    S_cnt = jnp.sum(pres)

    md = mu.reshape(P, K, D) - mu_pop_ref[...][None]
    msq = jnp.sum(jnp.sum(md * md, axis=2), axis=1, keepdims=True)
    mu_reg = jnp.sum(pres * msq)

    Ld = L.reshape(P, K, D, D) - Lp_ref[...][None]
    lsq = jnp.sum(jnp.sum(jnp.sum(Ld * Ld, axis=3), axis=2), axis=1,
                  keepdims=True)
    L_reg = jnp.sum(pres * lsq)

    reg = (S_cnt / _N_SUBJECTS) * (
        _LAMBDA_MU / 2.0 * mu_reg + _LAMBDA_L / 2.0 * L_reg)
    out_ref[0, 0] = nll + reg


def kernel(x, mu_pop, L_pop, mu_subj, L_subj, gamma, subject_ids):
    B, T, D = x.shape
    K = mu_pop.shape[0]
    P = mu_subj.shape[0]
    f32 = jnp.float32

    sid = subject_ids.astype(jnp.int32)
    xg = jnp.concatenate([x, jnp.ones((B, T, 1), f32), gamma], axis=2)
    xgT = jnp.swapaxes(xg, 1, 2)
    Ls = L_subj.reshape(P * K, D, D)
    mu_s = mu_subj.reshape(P * K, D)
    diag = jnp.diagonal(Ls, axis1=1, axis2=2)
    ids2 = sid.reshape(1, B)

    out = pl.pallas_call(
        _body,
        out_shape=jax.ShapeDtypeStruct((1, 1), f32),
        in_specs=[
            pl.BlockSpec(memory_space=pltpu.SMEM),
            pl.BlockSpec(memory_space=pltpu.VMEM),
            pl.BlockSpec(memory_space=pltpu.VMEM),
            pl.BlockSpec(memory_space=pltpu.VMEM),
            pl.BlockSpec(memory_space=pltpu.VMEM),
            pl.BlockSpec(memory_space=pltpu.VMEM),
            pl.BlockSpec(memory_space=pltpu.VMEM),
            pl.BlockSpec(memory_space=pltpu.VMEM),
        ],
        out_specs=pl.BlockSpec(memory_space=pltpu.SMEM),
        scratch_shapes=[
            pltpu.VMEM((P * K, D + 1, D + 1), f32),
        ],
    )(sid, ids2, xgT, mu_s, mu_pop, Ls, L_pop, diag)
    return out[0, 0]


# speedup vs baseline: 33.7957x; 1.2048x over previous
import jax
import jax.numpy as jnp
import numpy as np
from jax.experimental import pallas as pl
from jax.experimental.pallas import tpu as pltpu

_LAMBDA_MU = 0.1
_LAMBDA_L = 0.1
_N_SUBJECTS = 16
_LOG2PI = float(np.log(2.0 * np.pi))


def _body(sid_ref, ids_ref, xgT_ref, mu_subj_ref,
          mu_pop_ref, Ls_ref, Lp_ref, out_ref, ahat_scr):
    B, T, D, K, P = 16, 512, 32, 8, 16
    DA = D + 1
    f32 = jnp.float32

    L = Ls_ref[...]
    ii = jax.lax.broadcasted_iota(jnp.int32, (D, D), 0)
    jj = jax.lax.broadcasted_iota(jnp.int32, (D, D), 1)
    eye = (ii == jj).astype(f32)
    strict_lo = (ii > jj).astype(f32)

    d = jnp.sum(L * eye, axis=1)
    rinv = 1.0 / d
    N = -(L * strict_lo) * rinv[:, :, None]
    bmm = lambda a, b, dn: jax.lax.dot_general(
        a, b, (dn, ((0,), (0,))), preferred_element_type=f32)
    X = eye[None] + N
    Npow = N
    for _ in range(4):
        Npow = bmm(Npow, Npow, ((2,), (1,)))
        X = X + bmm(X, Npow, ((2,), (1,)))
    Linv = X * rinv[:, None, :]

    A = bmm(Linv, Linv, ((1,), (1,)))
    mu = mu_subj_ref[...]
    bvec = jnp.sum(A * mu[:, None, :], axis=2)
    cval = jnp.sum(mu * bvec, axis=1, keepdims=True)
    logdet = jnp.sum(jnp.log(d), axis=1, keepdims=True)
    Cval = (-0.5 * D * _LOG2PI) - logdet

    ahat_scr[:, 0:D, 0:D] = A
    ahat_scr[:, D:DA, 0:D] = -bvec[:, None, :]
    ahat_scr[:, 0:D, D:DA] = -bvec[:, :, None]
    ahat_scr[:, D:DA, D:DA] = (cval - 2.0 * Cval)[:, :, None]

    acc = jnp.zeros((), dtype=f32)
    for b in range(B):
        s = sid_ref[b]
        XG = xgT_ref[b]
        xT = XG[0:DA]
        gT = XG[DA:DA + K]
        Wg = gT[:, None, :] * xT[None]
        S = bmm(Wg, jnp.broadcast_to(xT[None], (K, DA, T)), ((2,), (2,)))
        Ah = ahat_scr[pl.ds(s * K, K)]
        acc = acc + jnp.sum(Ah * S)
    nll = 0.5 * acc / float(B * T)

    ids_v = ids_ref[...]
    pio = jax.lax.broadcasted_iota(jnp.int32, (P, B), 0)
    pres = jnp.max((pio == ids_v).astype(f32), axis=1, keepdims=True)
    S_cnt = jnp.sum(pres)

    md = mu.reshape(P, K, D) - mu_pop_ref[...][None]
    msq = jnp.sum(jnp.sum(md * md, axis=2), axis=1, keepdims=True)
    mu_reg = jnp.sum(pres * msq)

    Ld = L.reshape(P, K, D, D) - Lp_ref[...][None]
    lsq = jnp.sum(jnp.sum(jnp.sum(Ld * Ld, axis=3), axis=2), axis=1,
                  keepdims=True)
    L_reg = jnp.sum(pres * lsq)

    reg = (S_cnt / _N_SUBJECTS) * (
        _LAMBDA_MU / 2.0 * mu_reg + _LAMBDA_L / 2.0 * L_reg)
    out_ref[0, 0] = nll + reg


def kernel(x, mu_pop, L_pop, mu_subj, L_subj, gamma, subject_ids):
    B, T, D = x.shape
    K = mu_pop.shape[0]
    P = mu_subj.shape[0]
    f32 = jnp.float32

    sid = subject_ids.astype(jnp.int32)
    xg = jnp.concatenate([x, jnp.ones((B, T, 1), f32), gamma], axis=2)
    xgT = jnp.swapaxes(xg, 1, 2)
    Ls = L_subj.reshape(P * K, D, D)
    mu_s = mu_subj.reshape(P * K, D)
    ids2 = sid.reshape(1, B)

    out = pl.pallas_call(
        _body,
        out_shape=jax.ShapeDtypeStruct((1, 1), f32),
        in_specs=[
            pl.BlockSpec(memory_space=pltpu.SMEM),
            pl.BlockSpec(memory_space=pltpu.VMEM),
            pl.BlockSpec(memory_space=pltpu.VMEM),
            pl.BlockSpec(memory_space=pltpu.VMEM),
            pl.BlockSpec(memory_space=pltpu.VMEM),
            pl.BlockSpec(memory_space=pltpu.VMEM),
            pl.BlockSpec(memory_space=pltpu.VMEM),
        ],
        out_specs=pl.BlockSpec(memory_space=pltpu.SMEM),
        scratch_shapes=[
            pltpu.VMEM((P * K, D + 1, D + 1), f32),
        ],
    )(sid, ids2, xgT, mu_s, mu_pop, Ls, L_pop)
    return out[0, 0]
